# baseline (device time: 68281 ns/iter reference)
import jax
import jax.numpy as jnp
from jax import lax
from jax.experimental import pallas as pl
from jax.experimental.pallas import tpu as pltpu

N_DEV = 4
B = 2
S = 512
H = 8
Dh = 64
D_MODEL = 768
BLK = 64
NBLK = S // BLK

PERM = (0, 3, 6, 1, 4, 7, 2, 5)
GSTART = (0, 192, 384)
GLEN = (192, 192, 128)
GP = 192
SP = 3 * GP


def kernel(x, Wq, K_ext, V_ext, Wo):
    def body(x_ref, wq_ref, k_ref, v_ref, wo_ref, out_ref,
             comm_ref, acc_ref, ctx_ref, w_ref,
             send_sems, recv_sems):
        my = lax.axis_index("i")
        left = lax.rem(my + N_DEV - 1, N_DEV)
        right = lax.rem(my + 1, N_DEV)

        def q8(v):
            return jnp.clip(jnp.round(v * 32.0), -127.0, 127.0).astype(
                jnp.int8)

        for d in range(2):
            kd = k_ref[d].reshape(S, H * Dh)
            vd = v_ref[d].reshape(S, H * Dh)
            for beta in range(3):
                for m in range(3):
                    i = beta + 3 * m
                    dst = beta * GP + m * BLK
                    if i < NBLK:
                        comm_ref[d, 0, 0, pl.ds(dst, BLK), :] = q8(
                            kd[i * BLK:(i + 1) * BLK, :])
                        comm_ref[d, 0, 1, pl.ds(dst, BLK), :] = q8(
                            vd[i * BLK:(i + 1) * BLK, :])
                    else:
                        z = jnp.zeros((BLK, H * Dh), jnp.int8)
                        comm_ref[d, 0, 0, pl.ds(dst, BLK), :] = z
                        comm_ref[d, 0, 1, pl.ds(dst, BLK), :] = z

        barrier = pltpu.get_barrier_semaphore()
        for nbr in (left, right):
            pl.semaphore_signal(barrier, inc=1, device_id=(nbr,),
                                device_id_type=pl.DeviceIdType.MESH)
        pl.semaphore_wait(barrier, 2)

        k_rdmas = [[], []]
        v_rdmas = [[], []]

        def start_sub(h, kvi, rd):
            for d, tgt in ((0, right), (1, left)):
                r = pltpu.make_async_remote_copy(
                    src_ref=comm_ref.at[d, h, kvi],
                    dst_ref=comm_ref.at[d, h + 1, kvi],
                    send_sem=send_sems.at[d, kvi, h],
                    recv_sem=recv_sems.at[d, kvi, h],
                    device_id=(tgt,),
                    device_id_type=pl.DeviceIdType.MESH,
                )
                r.start()
                rd[d].append(r)

        start_sub(0, 0, k_rdmas)
        start_sub(0, 1, v_rdmas)

        xq = x_ref[...].reshape(B * S, D_MODEL).astype(jnp.bfloat16)
        wq = wq_ref[...].astype(jnp.bfloat16)
        q = jnp.dot(xq, wq, preferred_element_type=jnp.float32)
        q = (q * (0.125 / 32.0)).astype(jnp.bfloat16)
        q_g = jnp.concatenate(
            [q[b * S + j * BLK: b * S + (j + 1) * BLK, :]
             for b in range(B) for j in PERM], axis=0)

        r_row = lax.broadcasted_iota(jnp.int32, (S, 1), 0)
        a_row = r_row // GP
        j_row = a_row + 3 * ((r_row - a_row * GP) // BLK)
        qb_g = my * NBLK + j_row
        qr_g = lax.rem(qb_g, 3)

        pp = lax.broadcasted_iota(jnp.int32, (1, SP), 1)
        bb_col = pp // GP
        i_col = bb_col + 3 * ((pp - bb_col * GP) // BLK)
        kb0_col = my * NBLK + i_col
        mask0 = ((qb_g == kb0_col) | (kb0_col == 0)
                 | (lax.rem(qb_g + kb0_col, 3) == 0)) & (i_col < NBLK)

        m_col = lax.broadcasted_iota(jnp.int32, (1, GP), 1) // BLK

        def beta_for(o, a):
            return lax.rem(lax.rem(-(my + o) * NBLK - a, 3) + 3, 3)

        den = [[[None] * 3 for _ in range(H)] for _ in range(B)]
        for h in range(N_DEV):
            if h > 0:
                k_rdmas[0][h - 1].wait()
                k_rdmas[1][h - 1].wait()
                if h < N_DEV - 1:
                    start_sub(h, 0, k_rdmas)

            if h == 0:
                for d in range(2):
                    b = d
                    for hd in range(H):
                        hs = slice(hd * Dh, (hd + 1) * Dh)
                        qbh = q_g[b * S:(b + 1) * S, hs]
                        kbh = comm_ref[d, 0, 0, :, hs].astype(jnp.bfloat16)
                        s = lax.dot_general(
                            qbh, kbh, (((1,), (1,)), ((), ())),
                            preferred_element_type=jnp.float32)
                        w = jnp.where(mask0, jnp.exp(s.astype(jnp.bfloat16)),
                                      jnp.bfloat16(0.0))
                        vbh = comm_ref[d, 0, 1, :, hs].astype(jnp.bfloat16)
                        pv = jnp.dot(w, vbh,
                                     preferred_element_type=jnp.float32)
                        acc_ref[pl.ds(b * S, S), pl.ds(hd * Dh, Dh)] = pv
                        ds_full = jnp.sum(w.astype(jnp.float32), axis=1,
                                          keepdims=True)
                        for a in range(3):
                            den[b][hd][a] = ds_full[
                                GSTART[a]:GSTART[a] + GLEN[a], :]
                continue

            origins = (lax.rem(my - h + N_DEV, N_DEV),
                       lax.rem(my + h, N_DEV))
            for d in range(2):
                b = d
                for a in range(3):
                    t = beta_for(origins[d], a)
                    colmask = (t != 2) | (m_col < 2)
                    for hd in range(H):
                        hs = slice(hd * Dh, (hd + 1) * Dh)
                        qa = q_g[b * S + GSTART[a]:
                                 b * S + GSTART[a] + GLEN[a], hs]
                        kbh = comm_ref[d, h, 0, pl.ds(t * GP, GP),
                                       hs].astype(jnp.bfloat16)
                        s = lax.dot_general(
                            qa, kbh, (((1,), (1,)), ((), ())),
                            preferred_element_type=jnp.float32)
                        w = jnp.where(colmask,
                                      jnp.exp(s.astype(jnp.bfloat16)),
                                      jnp.bfloat16(0.0))
                        w_ref[d, pl.ds(GSTART[a], GLEN[a]),
                              pl.ds(hd * GP, GP)] = w
                        ds = jnp.sum(w.astype(jnp.float32), axis=1,
                                     keepdims=True)
                        den[b][hd][a] = den[b][hd][a] + ds

            v_rdmas[0][h - 1].wait()
            v_rdmas[1][h - 1].wait()
            if h < N_DEV - 1:
                start_sub(h, 1, v_rdmas)
            for d in range(2):
                b = d
                for a in range(3):
                    t = beta_for(origins[d], a)
                    for hd in range(H):
                        wv = w_ref[d, pl.ds(GSTART[a], GLEN[a]),
                                   pl.ds(hd * GP, GP)]
                        vbh = comm_ref[d, h, 1, pl.ds(t * GP, GP),
                                       hd * Dh:(hd + 1) * Dh].astype(
                                           jnp.bfloat16)
                        pv = jnp.dot(wv, vbh,
                                     preferred_element_type=jnp.float32)
                        acc_ref[pl.ds(b * S + GSTART[a], GLEN[a]),
                                pl.ds(hd * Dh, Dh)] += pv

        cond0 = (qr_g != 0) & (my != 0)
        slots0 = (my, lax.rem(N_DEV - my, N_DEV))
        b0kv = []
        for d in range(2):
            sel = []
            for kvi in range(2):
                pick = jnp.zeros((BLK, H * Dh), jnp.bfloat16)
                for s_ in range(N_DEV):
                    blk = comm_ref[d, s_, kvi, pl.ds(0, BLK), :].astype(
                        jnp.bfloat16)
                    pick = jnp.where(slots0[d] == s_, blk, pick)
                sel.append(pick)
            b0kv.append(sel)
        for d in range(2):
            b = d
            for hd in range(H):
                hs = slice(hd * Dh, (hd + 1) * Dh)
                b0k = b0kv[d][0][:, hs]
                s0 = lax.dot_general(
                    q_g[b * S:(b + 1) * S, hs], b0k,
                    (((1,), (1,)), ((), ())),
                    preferred_element_type=jnp.float32)
                w0 = jnp.where(cond0, jnp.exp(s0.astype(jnp.bfloat16)),
                               jnp.bfloat16(0.0))
                b0v = b0kv[d][1][:, hs]
                pv0 = jnp.dot(w0, b0v, preferred_element_type=jnp.float32)
                acc_ref[pl.ds(b * S, S), pl.ds(hd * Dh, Dh)] += pv0
                ds0 = jnp.sum(w0.astype(jnp.float32), axis=1, keepdims=True)
                for a in range(3):
                    den[b][hd][a] = den[b][hd][a] + ds0[
                        GSTART[a]:GSTART[a] + GLEN[a], :]

        for b in range(B):
            for hd in range(H):
                dfull = jnp.concatenate(
                    [den[b][hd][0], den[b][hd][1], den[b][hd][2]], axis=0)
                piece = acc_ref[pl.ds(b * S, S), pl.ds(hd * Dh, Dh)]
                ctx_ref[pl.ds(b * S, S), pl.ds(hd * Dh, Dh)] = (
                    piece / (dfull * 32.0)).astype(jnp.bfloat16)

        out_g = jnp.dot(ctx_ref[...], wo_ref[...].astype(jnp.bfloat16),
                        preferred_element_type=jnp.float32)
        for b in range(B):
            for idx, j in enumerate(PERM):
                out_ref[b, j * BLK:(j + 1) * BLK, :] = out_g[
                    b * S + idx * BLK: b * S + (idx + 1) * BLK, :]

    return pl.pallas_call(
        body,
        out_shape=jax.ShapeDtypeStruct((B, S, D_MODEL), jnp.float32),
        in_specs=[pl.BlockSpec(memory_space=pltpu.VMEM)] * 5,
        out_specs=pl.BlockSpec(memory_space=pltpu.VMEM),
        scratch_shapes=[
            pltpu.VMEM((2, N_DEV, 2, SP, H * Dh), jnp.int8),
            pltpu.VMEM((B * S, H * Dh), jnp.float32),
            pltpu.VMEM((B * S, H * Dh), jnp.bfloat16),
            pltpu.VMEM((2, S, H * GP), jnp.bfloat16),
            pltpu.SemaphoreType.DMA((2, 2, N_DEV - 1)),
            pltpu.SemaphoreType.DMA((2, 2, N_DEV - 1)),
        ],
        compiler_params=pltpu.CompilerParams(
            collective_id=0, vmem_limit_bytes=60 * 1024 * 1024),
    )(x, Wq, K_ext, V_ext, Wo)


# device time: 49266 ns/iter; 1.3860x vs baseline; 1.3860x over previous
import jax
import jax.numpy as jnp
from jax import lax
from jax.experimental import pallas as pl
from jax.experimental.pallas import tpu as pltpu

N_DEV = 4
B = 2
S = 512
H = 8
Dh = 64
D_MODEL = 768
BLK = 64


def kernel(x, Wq, K_ext, V_ext, Wo):
    def body(x_ref, wq_ref, k_ref, v_ref, wo_ref, out_ref,
             comm_ref, acc_ref, ctx_ref, w_ref,
             send_sems, recv_sems):
        my = lax.axis_index("i")
        left = lax.rem(my + N_DEV - 1, N_DEV)
        right = lax.rem(my + 1, N_DEV)

        for d in range(2):
            comm_ref[d, 0, 0] = jnp.clip(
                jnp.round(k_ref[d].reshape(S, H * Dh) * 32.0),
                -127.0, 127.0).astype(jnp.int8)
            comm_ref[d, 0, 1] = jnp.clip(
                jnp.round(v_ref[d].reshape(S, H * Dh) * 32.0),
                -127.0, 127.0).astype(jnp.int8)

        barrier = pltpu.get_barrier_semaphore()
        for nbr in (left, right):
            pl.semaphore_signal(barrier, inc=1, device_id=(nbr,),
                                device_id_type=pl.DeviceIdType.MESH)
        pl.semaphore_wait(barrier, 2)

        k_rdmas = [[], []]
        v_rdmas = [[], []]

        def start_sub(h, kvi, rd):
            for d, tgt in ((0, right), (1, left)):
                r = pltpu.make_async_remote_copy(
                    src_ref=comm_ref.at[d, h, kvi],
                    dst_ref=comm_ref.at[d, h + 1, kvi],
                    send_sem=send_sems.at[d, kvi, h],
                    recv_sem=recv_sems.at[d, kvi, h],
                    device_id=(tgt,),
                    device_id_type=pl.DeviceIdType.MESH,
                )
                r.start()
                rd[d].append(r)

        start_sub(0, 0, k_rdmas)
        start_sub(0, 1, v_rdmas)

        xq = x_ref[...].reshape(B * S, D_MODEL).astype(jnp.bfloat16)
        wq = wq_ref[...].astype(jnp.bfloat16)
        q = jnp.dot(xq, wq, preferred_element_type=jnp.float32)
        q = (q * (0.125 / 32.0)).astype(jnp.bfloat16)

        qblk = lax.broadcasted_iota(jnp.int32, (S, 1), 0) // BLK
        qb_g = my * (S // BLK) + qblk
        qr_need = lax.rem(3 - lax.rem(qb_g, 3), 3)
        cblk = lax.broadcasted_iota(jnp.int32, (1, S), 1) // BLK

        masks = {}

        def mask_for(k):
            if k not in masks:
                origin = lax.rem(my - k + N_DEV, N_DEV)
                kb_g = origin * (S // BLK) + cblk
                kr = lax.rem(kb_g, 3)
                masks[k] = (qb_g == kb_g) | (kb_g == 0) | (kr == qr_need)
            return masks[k]

        den = [[None] * H for _ in range(B)]
        for h in range(N_DEV):
            if h > 0:
                k_rdmas[0][h - 1].wait()
                k_rdmas[1][h - 1].wait()
                if h < N_DEV - 1:
                    start_sub(h, 0, k_rdmas)
            for d in range(2):
                b = d
                mask = mask_for(h if d == 0 else (N_DEV - h) % N_DEV)
                for hd in range(H):
                    qbh = q[b * S:(b + 1) * S, hd * Dh:(hd + 1) * Dh]
                    kbh = comm_ref[d, h, 0, :, hd * Dh:(hd + 1) * Dh].astype(
                        jnp.bfloat16)
                    s = lax.dot_general(
                        qbh, kbh, (((1,), (1,)), ((), ())),
                        preferred_element_type=jnp.float32)
                    w = jnp.where(mask, jnp.exp(s.astype(jnp.bfloat16)),
                                  jnp.bfloat16(0.0))
                    w_ref[d, :, pl.ds(hd * S, S)] = w
                    dsum = jnp.sum(w.astype(jnp.float32), axis=1,
                                   keepdims=True)
                    den[b][hd] = dsum if h == 0 else den[b][hd] + dsum

            if h > 0:
                v_rdmas[0][h - 1].wait()
                v_rdmas[1][h - 1].wait()
                if h < N_DEV - 1:
                    start_sub(h, 1, v_rdmas)
            for d in range(2):
                b = d
                for hd in range(H):
                    wv = w_ref[d, :, pl.ds(hd * S, S)]
                    vbh = comm_ref[d, h, 1, :, hd * Dh:(hd + 1) * Dh].astype(
                        jnp.bfloat16)
                    pv = jnp.dot(wv, vbh, preferred_element_type=jnp.float32)
                    if h == 0:
                        acc_ref[pl.ds(b * S, S), pl.ds(hd * Dh, Dh)] = pv
                    else:
                        acc_ref[pl.ds(b * S, S), pl.ds(hd * Dh, Dh)] += pv

        for b in range(B):
            for hd in range(H):
                piece = acc_ref[pl.ds(b * S, S), pl.ds(hd * Dh, Dh)]
                ctx_ref[pl.ds(b * S, S), pl.ds(hd * Dh, Dh)] = (
                    piece / (den[b][hd] * 32.0)).astype(jnp.bfloat16)

        out = jnp.dot(ctx_ref[...], wo_ref[...].astype(jnp.bfloat16),
                      preferred_element_type=jnp.float32)
        out_ref[...] = out.reshape(B, S, D_MODEL)

    return pl.pallas_call(
        body,
        out_shape=jax.ShapeDtypeStruct((B, S, D_MODEL), jnp.float32),
        in_specs=[pl.BlockSpec(memory_space=pltpu.VMEM)] * 5,
        out_specs=pl.BlockSpec(memory_space=pltpu.VMEM),
        scratch_shapes=[
            pltpu.VMEM((2, N_DEV, 2, S, H * Dh), jnp.int8),
            pltpu.VMEM((B * S, H * Dh), jnp.float32),
            pltpu.VMEM((B * S, H * Dh), jnp.bfloat16),
            pltpu.VMEM((2, S, H * S), jnp.bfloat16),
            pltpu.SemaphoreType.DMA((2, 2, N_DEV - 1)),
            pltpu.SemaphoreType.DMA((2, 2, N_DEV - 1)),
        ],
        compiler_params=pltpu.CompilerParams(
            collective_id=0, vmem_limit_bytes=60 * 1024 * 1024),
    )(x, Wq, K_ext, V_ext, Wo)
